# Initial kernel scaffold; baseline (speedup 1.0000x reference)
import functools

import jax
import jax.numpy as jnp
from jax import lax
from jax.experimental import pallas as pl
from jax.experimental.pallas import tpu as pltpu
from jax.experimental.pallas import tpu_sc as plsc

N = 10000
E = 320000
D = 128

NC = 2
NS = 16
NW = NC * NS
EPT = E // NW
C = 100
NCH = EPT // C
RPT = N // NS
ZR = 125

_mesh = plsc.VectorSubcoreMesh(core_axis_name="c", subcore_axis_name="s")


def _sc_degree(dst2d, ew2d):
    @functools.partial(
        pl.kernel,
        out_type=jax.ShapeDtypeStruct((NC, N), jnp.float32),
        mesh=_mesh,
        scratch_types=[
            pltpu.VMEM((NCH, C), jnp.int32),
            pltpu.VMEM((NCH, C), jnp.float32),
            pltpu.VMEM((2000,), jnp.float32),
            pltpu.VMEM_SHARED((N,), jnp.float32),
            pltpu.SemaphoreType.DMA,
        ],
    )
    def k(dst_hbm, ew_hbm, out_hbm, didx, wbuf, zbuf, acc, sem):
        cid = lax.axis_index("c")
        sid = lax.axis_index("s")
        wid = sid * NC + cid

        pltpu.sync_copy(dst_hbm.at[pl.ds(wid * NCH, NCH)], didx)
        pltpu.sync_copy(ew_hbm.at[pl.ds(wid * NCH, NCH)], wbuf)

        @pl.loop(0, 2000 // 16)
        def _(i):
            zbuf[pl.ds(i * 16, 16)] = jnp.zeros((16,), jnp.float32)

        @pl.when(sid == 0)
        def _():
            @pl.loop(0, N // 2000)
            def _(j):
                pltpu.sync_copy(zbuf, acc.at[pl.ds(j * 2000, 2000)])

        plsc.subcore_barrier()

        @pl.loop(0, NCH)
        def _(g):
            pltpu.sync_copy(wbuf.at[g], acc.at[didx.at[g]], add=True)

        plsc.subcore_barrier()

        @pl.when(sid == 0)
        def _():
            pltpu.sync_copy(acc, out_hbm.at[cid])

    return k(dst2d, ew2d)


def _sc_scatter(ys, src2d, dst2d, ew2d):
    @functools.partial(
        pl.kernel,
        out_type=jax.ShapeDtypeStruct((NC, N, D), jnp.float32),
        mesh=_mesh,
        scratch_types=[
            pltpu.VMEM((NCH, C), jnp.int32),
            pltpu.VMEM((NCH, C), jnp.int32),
            pltpu.VMEM((NCH, C), jnp.float32),
            pltpu.VMEM((C, D), jnp.float32),
            pltpu.VMEM((ZR, D), jnp.float32),
            pltpu.VMEM_SHARED((N, D), jnp.float32),
            pltpu.SemaphoreType.DMA,
        ],
    )
    def k(ys_hbm, src_hbm, dst_hbm, ew_hbm, out_hbm,
          sidx, didx, wbuf, rows, zbuf, acc, gsem):
        cid = lax.axis_index("c")
        sid = lax.axis_index("s")
        wid = sid * NC + cid

        pltpu.sync_copy(src_hbm.at[pl.ds(wid * NCH, NCH)], sidx)
        pltpu.sync_copy(dst_hbm.at[pl.ds(wid * NCH, NCH)], didx)
        pltpu.sync_copy(ew_hbm.at[pl.ds(wid * NCH, NCH)], wbuf)

        @pl.loop(0, ZR)
        def _(r):
            for j in range(D // 16):
                zbuf[r, pl.ds(j * 16, 16)] = jnp.zeros((16,), jnp.float32)

        @pl.loop(0, RPT // ZR)
        def _(j):
            pltpu.sync_copy(zbuf, acc.at[pl.ds(sid * RPT + j * ZR, ZR)])

        plsc.subcore_barrier()

        @pl.loop(0, NCH)
        def _(g):
            pltpu.async_copy(ys_hbm.at[sidx.at[g]], rows, gsem).wait()

            @pl.loop(0, C)
            def _(c):
                wv = jnp.full((16,), wbuf[g, c], jnp.float32)
                for j in range(D // 16):
                    rows[c, pl.ds(j * 16, 16)] = rows[c, pl.ds(j * 16, 16)] * wv

            pltpu.sync_copy(rows, acc.at[didx.at[g]], add=True)

        plsc.subcore_barrier()

        pltpu.sync_copy(acc.at[pl.ds(sid * RPT, RPT)],
                        out_hbm.at[cid, pl.ds(sid * RPT, RPT)])

    return k(ys, src2d, dst2d, ew2d)


_ROWS_BLK = 1000
_GRID = N // _ROWS_BLK


def _rows_spec():
    return pl.BlockSpec((_ROWS_BLK, D), lambda i: (i, 0))


def _full_spec(shape):
    return pl.BlockSpec(shape, lambda i: tuple(0 for _ in shape))


def _tc_dis(degT):
    def body(dp_ref, o_ref):
        deg = dp_ref[...][:, 0:1] + dp_ref[...][:, 1:2] + 1.0
        o_ref[...] = jnp.broadcast_to(lax.rsqrt(deg), o_ref.shape)

    return pl.pallas_call(
        body,
        grid=(_GRID,),
        in_specs=[pl.BlockSpec((_ROWS_BLK, 2), lambda i: (i, 0))],
        out_specs=_rows_spec(),
        out_shape=jax.ShapeDtypeStruct((N, D), jnp.float32),
    )(degT)


def _tc_first(x, W, dis):
    def body(x_ref, w_ref, dis_ref, o_ref):
        o_ref[...] = dis_ref[...] * jnp.dot(
            x_ref[...], w_ref[...], preferred_element_type=jnp.float32)

    return pl.pallas_call(
        body,
        grid=(_GRID,),
        in_specs=[_rows_spec(), _full_spec((D, D)), _rows_spec()],
        out_specs=_rows_spec(),
        out_shape=jax.ShapeDtypeStruct((N, D), jnp.float32),
    )(x, W, dis)


def _tc_mid(p0, p1, ys, dis, b, W):
    def body(p0_ref, p1_ref, ys_ref, dis_ref, b_ref, w_ref, o_ref):
        t = dis_ref[...] * (p0_ref[...] + p1_ref[...] + ys_ref[...]) + b_ref[...]
        h = jnp.where(t > 0, t, jnp.expm1(t))
        o_ref[...] = dis_ref[...] * jnp.dot(
            h, w_ref[...], preferred_element_type=jnp.float32)

    return pl.pallas_call(
        body,
        grid=(_GRID,),
        in_specs=[_rows_spec(), _rows_spec(), _rows_spec(), _rows_spec(),
                  _full_spec((1, D)), _full_spec((D, D))],
        out_specs=_rows_spec(),
        out_shape=jax.ShapeDtypeStruct((N, D), jnp.float32),
    )(p0, p1, ys, dis, b, W)


def _tc_last(p0, p1, ys, dis, b):
    def body(p0_ref, p1_ref, ys_ref, dis_ref, b_ref, o_ref):
        t = dis_ref[...] * (p0_ref[...] + p1_ref[...] + ys_ref[...]) + b_ref[...]
        nrm = jnp.sqrt(jnp.sum(t * t, axis=1, keepdims=True))
        o_ref[...] = t / jnp.maximum(nrm, 1e-12)

    return pl.pallas_call(
        body,
        grid=(_GRID,),
        in_specs=[_rows_spec(), _rows_spec(), _rows_spec(), _rows_spec(),
                  _full_spec((1, D))],
        out_specs=_rows_spec(),
        out_shape=jax.ShapeDtypeStruct((N, D), jnp.float32),
    )(p0, p1, ys, dis, b)


def kernel(x, edge_index, edge_weight, W1, b1, W2, b2, W3, b3):
    src2d = edge_index[0].reshape(E // C, C)
    dst2d = edge_index[1].reshape(E // C, C)
    ew2d = edge_weight.reshape(E // C, C)

    degp = _sc_degree(dst2d, ew2d)
    dis = _tc_dis(degp.T)

    ys = _tc_first(x, W1, dis)
    for b, Wn in ((b1, W2), (b2, W3)):
        p = _sc_scatter(ys, src2d, dst2d, ew2d)
        ys = _tc_mid(p[0], p[1], ys, dis, b.reshape(1, D), Wn)
    p = _sc_scatter(ys, src2d, dst2d, ew2d)
    return _tc_last(p[0], p[1], ys, dis, b3.reshape(1, D))


# speedup vs baseline: 14.3081x; 14.3081x over previous
import functools

import jax
import jax.numpy as jnp
from jax import lax
from jax.experimental import pallas as pl
from jax.experimental.pallas import tpu as pltpu
from jax.experimental.pallas import tpu_sc as plsc

N = 10000
E = 320000
D = 128

NC = 2
NS = 16
NW = NC * NS
EPT = E // NW
C = 125
NCH = EPT // C
GRP = 8
NG = NCH // GRP
WPAD = 128
NP = 10240
RPT = NP // NS

_mesh = plsc.VectorSubcoreMesh(core_axis_name="c", subcore_axis_name="s")


def _sc_degree(dst3d, ew3d):
    @functools.partial(
        pl.kernel,
        out_type=jax.ShapeDtypeStruct((NC * NP,), jnp.float32),
        mesh=_mesh,
        scratch_types=[
            pltpu.VMEM((NCH, C), jnp.int32),
            pltpu.VMEM((NCH, C), jnp.float32),
            pltpu.VMEM((2048,), jnp.float32),
            pltpu.VMEM_SHARED((NP,), jnp.float32),
            pltpu.SemaphoreType.DMA,
        ],
    )
    def k(dst_hbm, ew_hbm, out_hbm, didx, wbuf, zbuf, acc, sem):
        cid = lax.axis_index("c")
        sid = lax.axis_index("s")
        wid = sid * NC + cid

        pltpu.sync_copy(dst_hbm.at[wid], didx)
        pltpu.sync_copy(ew_hbm.at[wid], wbuf)

        @pl.loop(0, 2048 // 16)
        def _(i):
            zbuf[pl.ds(i * 16, 16)] = jnp.zeros((16,), jnp.float32)

        @pl.when(sid == 0)
        def _():
            @pl.loop(0, NP // 2048)
            def _(j):
                pltpu.sync_copy(zbuf, acc.at[pl.ds(j * 2048, 2048)])

        plsc.subcore_barrier()

        @pl.loop(0, NCH)
        def _(g):
            pltpu.sync_copy(wbuf.at[g], acc.at[didx.at[g]], add=True)

        plsc.subcore_barrier()

        @pl.when(sid == 0)
        def _():
            pltpu.sync_copy(acc, out_hbm.at[pl.ds(cid * NP, NP)])

    return k(dst3d, ew3d)


def _sc_scatter(ys, src3d, dst3d, ewp3d):
    @functools.partial(
        pl.kernel,
        out_type=jax.ShapeDtypeStruct((NC, NP, D), jnp.float32),
        mesh=_mesh,
        scratch_types=[
            pltpu.VMEM((GRP, C), jnp.int32),
            pltpu.VMEM((GRP, C), jnp.int32),
            pltpu.VMEM((GRP, WPAD), jnp.float32),
            pltpu.VMEM((128, D), jnp.float32),
            pltpu.VMEM_SHARED((NP, D), jnp.float32),
            pltpu.SemaphoreType.DMA,
        ],
    )
    def k(ys_hbm, src_hbm, dst_hbm, ewp_hbm, out_hbm,
          sidx, didx, wbuf, rows, acc, gsem):
        cid = lax.axis_index("c")
        sid = lax.axis_index("s")
        wid = sid * NC + cid

        @pl.loop(0, 128)
        def _(r):
            for j in range(D // 16):
                rows[r, pl.ds(j * 16, 16)] = jnp.zeros((16,), jnp.float32)

        @pl.loop(0, RPT // 128)
        def _(j):
            pltpu.sync_copy(rows, acc.at[pl.ds(sid * RPT + j * 128, 128)])

        plsc.subcore_barrier()

        @pl.loop(0, NG)
        def _(gg):
            pltpu.sync_copy(src_hbm.at[wid, pl.ds(gg * GRP, GRP)], sidx)
            pltpu.sync_copy(dst_hbm.at[wid, pl.ds(gg * GRP, GRP)], didx)
            pltpu.sync_copy(ewp_hbm.at[wid, pl.ds(gg * GRP, GRP)], wbuf)

            @pl.loop(0, GRP)
            def _(g):
                pltpu.async_copy(ys_hbm.at[sidx.at[g]],
                                 rows.at[pl.ds(0, C)], gsem).wait()

                for cv in range(WPAD // 16):
                    w16 = wbuf[g, pl.ds(cv * 16, 16)]
                    for l in range(16):
                        wv = jnp.full((16,), w16[l], jnp.float32)
                        c = cv * 16 + l
                        for j in range(D // 16):
                            rows[c, pl.ds(j * 16, 16)] = (
                                rows[c, pl.ds(j * 16, 16)] * wv)

                pltpu.sync_copy(rows.at[pl.ds(0, C)],
                                acc.at[didx.at[g]], add=True)

        plsc.subcore_barrier()

        pltpu.sync_copy(acc.at[pl.ds(sid * RPT, RPT)],
                        out_hbm.at[cid, pl.ds(sid * RPT, RPT)])

    return k(ys, src3d, dst3d, ewp3d)


_ROWS_BLK = 1000
_GRID = N // _ROWS_BLK


def _rows_spec():
    return pl.BlockSpec((_ROWS_BLK, D), lambda i: (i, 0))


def _full_spec(shape):
    return pl.BlockSpec(shape, lambda i: tuple(0 for _ in shape))


def _tc_dis(degT):
    def body(dp_ref, o_ref):
        deg = dp_ref[...][:, 0:1] + dp_ref[...][:, 1:2] + 1.0
        o_ref[...] = jnp.broadcast_to(lax.rsqrt(deg), o_ref.shape)

    return pl.pallas_call(
        body,
        grid=(_GRID,),
        in_specs=[pl.BlockSpec((_ROWS_BLK, 2), lambda i: (i, 0))],
        out_specs=_rows_spec(),
        out_shape=jax.ShapeDtypeStruct((N, D), jnp.float32),
    )(degT)


def _tc_first(x, W, dis):
    def body(x_ref, w_ref, dis_ref, o_ref):
        o_ref[...] = dis_ref[...] * jnp.dot(
            x_ref[...], w_ref[...], preferred_element_type=jnp.float32)

    return pl.pallas_call(
        body,
        grid=(_GRID,),
        in_specs=[_rows_spec(), _full_spec((D, D)), _rows_spec()],
        out_specs=_rows_spec(),
        out_shape=jax.ShapeDtypeStruct((N, D), jnp.float32),
    )(x, W, dis)


def _tc_mid(p0, p1, ys, dis, b, W):
    def body(p0_ref, p1_ref, ys_ref, dis_ref, b_ref, w_ref, o_ref):
        t = dis_ref[...] * (p0_ref[...] + p1_ref[...] + ys_ref[...]) + b_ref[...]
        h = jnp.where(t > 0, t, jnp.exp(t) - 1.0)
        o_ref[...] = dis_ref[...] * jnp.dot(
            h, w_ref[...], preferred_element_type=jnp.float32)

    return pl.pallas_call(
        body,
        grid=(_GRID,),
        in_specs=[_rows_spec(), _rows_spec(), _rows_spec(), _rows_spec(),
                  _full_spec((1, D)), _full_spec((D, D))],
        out_specs=_rows_spec(),
        out_shape=jax.ShapeDtypeStruct((N, D), jnp.float32),
    )(p0, p1, ys, dis, b, W)


def _tc_last(p0, p1, ys, dis, b):
    def body(p0_ref, p1_ref, ys_ref, dis_ref, b_ref, o_ref):
        t = dis_ref[...] * (p0_ref[...] + p1_ref[...] + ys_ref[...]) + b_ref[...]
        nrm = jnp.sqrt(jnp.sum(t * t, axis=1, keepdims=True))
        o_ref[...] = t / jnp.maximum(nrm, 1e-12)

    return pl.pallas_call(
        body,
        grid=(_GRID,),
        in_specs=[_rows_spec(), _rows_spec(), _rows_spec(), _rows_spec(),
                  _full_spec((1, D))],
        out_specs=_rows_spec(),
        out_shape=jax.ShapeDtypeStruct((N, D), jnp.float32),
    )(p0, p1, ys, dis, b)


def kernel(x, edge_index, edge_weight, W1, b1, W2, b2, W3, b3):
    src3d = edge_index[0].reshape(NW, NCH, C)
    dst3d = edge_index[1].reshape(NW, NCH, C)
    ew3d = edge_weight.reshape(NW, NCH, C)
    ewp3d = jnp.zeros((NW, NCH, WPAD), jnp.float32).at[:, :, :C].set(ew3d)

    degp = _sc_degree(dst3d, ew3d).reshape(NC, NP)
    dis = _tc_dis(degp.T)

    ys = _tc_first(x, W1, dis)
    for b, Wn in ((b1, W2), (b2, W3)):
        p = _sc_scatter(ys, src3d, dst3d, ewp3d)
        ys = _tc_mid(p[0], p[1], ys, dis, b.reshape(1, D), Wn)
    p = _sc_scatter(ys, src3d, dst3d, ewp3d)
    return _tc_last(p[0], p[1], ys, dis, b3.reshape(1, D))


# speedup vs baseline: 19.8483x; 1.3872x over previous
import functools

import jax
import jax.numpy as jnp
from jax import lax
from jax.experimental import pallas as pl
from jax.experimental.pallas import tpu as pltpu
from jax.experimental.pallas import tpu_sc as plsc

N = 10000
E = 320000
D = 128

NC = 2
NS = 16
NW = NC * NS
EPT = E // NW

DC = 125
DNCH = EPT // DC

C = 50
NCHT = EPT // C
GRP = 40
NG = NCHT // GRP
NB = 4
RB = 64
WPAD = 128
NP = 10240
RPT = NP // NS

_mesh = plsc.VectorSubcoreMesh(core_axis_name="c", subcore_axis_name="s")


def _sc_degree(dst3d, ew3d):
    @functools.partial(
        pl.kernel,
        out_type=jax.ShapeDtypeStruct((NC * NP,), jnp.float32),
        mesh=_mesh,
        scratch_types=[
            pltpu.VMEM((DNCH, DC), jnp.int32),
            pltpu.VMEM((DNCH, DC), jnp.float32),
            pltpu.VMEM((2048,), jnp.float32),
            pltpu.VMEM_SHARED((NP,), jnp.float32),
            pltpu.SemaphoreType.DMA,
        ],
    )
    def k(dst_hbm, ew_hbm, out_hbm, didx, wbuf, zbuf, acc, sem):
        cid = lax.axis_index("c")
        sid = lax.axis_index("s")
        wid = sid * NC + cid

        pltpu.sync_copy(dst_hbm.at[wid], didx)
        pltpu.sync_copy(ew_hbm.at[wid], wbuf)

        @pl.loop(0, 2048 // 16)
        def _(i):
            zbuf[pl.ds(i * 16, 16)] = jnp.zeros((16,), jnp.float32)

        @pl.when(sid == 0)
        def _():
            @pl.loop(0, NP // 2048)
            def _(j):
                pltpu.sync_copy(zbuf, acc.at[pl.ds(j * 2048, 2048)])

        plsc.subcore_barrier()

        @pl.loop(0, DNCH // 16)
        def _(gq):
            for kk in range(16):
                pltpu.async_copy(wbuf.at[gq * 16 + kk],
                                 acc.at[didx.at[gq * 16 + kk]], sem,
                                 add=True)
            for kk in range(16):
                pltpu.make_async_copy(wbuf.at[gq * 16 + kk],
                                      acc.at[didx.at[gq * 16 + kk]],
                                      sem).wait()

        plsc.subcore_barrier()

        @pl.when(sid == 0)
        def _():
            pltpu.sync_copy(acc, out_hbm.at[pl.ds(cid * NP, NP)])

    return k(dst3d, ew3d)


def _sc_scatter(ys, src3d, dst3d, ewp3d):
    @functools.partial(
        pl.kernel,
        out_type=jax.ShapeDtypeStruct((NC, NP, D), jnp.float32),
        mesh=_mesh,
        scratch_types=[
            pltpu.VMEM((GRP, C), jnp.int32),
            pltpu.VMEM((GRP, C), jnp.int32),
            pltpu.VMEM((GRP, WPAD), jnp.float32),
            pltpu.VMEM((RB, D), jnp.float32),
            pltpu.VMEM((RB, D), jnp.float32),
            pltpu.VMEM((RB, D), jnp.float32),
            pltpu.VMEM((RB, D), jnp.float32),
            pltpu.VMEM_SHARED((NP, D), jnp.float32),
            pltpu.SemaphoreType.DMA,
            pltpu.SemaphoreType.DMA,
            pltpu.SemaphoreType.DMA,
            pltpu.SemaphoreType.DMA,
            pltpu.SemaphoreType.DMA,
            pltpu.SemaphoreType.DMA,
            pltpu.SemaphoreType.DMA,
            pltpu.SemaphoreType.DMA,
            pltpu.SemaphoreType.DMA,
        ],
    )
    def k(ys_hbm, src_hbm, dst_hbm, ewp_hbm, out_hbm,
          sidx, didx, wbuf, r0, r1, r2, r3, acc,
          g0, g1, g2, g3, s0, s1, s2, s3, zsem):
        cid = lax.axis_index("c")
        sid = lax.axis_index("s")
        wid = sid * NC + cid
        rows = (r0, r1, r2, r3)
        gsem = (g0, g1, g2, g3)
        ssem = (s0, s1, s2, s3)

        def gather(i, b):
            return pltpu.async_copy(ys_hbm.at[sidx.at[i]],
                                    rows[b].at[pl.ds(0, C)], gsem[b])

        def gather_wait(i, b):
            pltpu.make_async_copy(ys_hbm.at[sidx.at[i]],
                                  rows[b].at[pl.ds(0, C)], gsem[b]).wait()

        def scatter(i, b):
            return pltpu.async_copy(rows[b].at[pl.ds(0, C)],
                                    acc.at[didx.at[i]], ssem[b], add=True)

        def scatter_wait(i, b):
            pltpu.make_async_copy(rows[b].at[pl.ds(0, C)],
                                  acc.at[didx.at[i]], ssem[b]).wait()

        @pl.loop(0, RB)
        def _(r):
            for j in range(D // 16):
                r0[r, pl.ds(j * 16, 16)] = jnp.zeros((16,), jnp.float32)

        for t in range(RPT // RB):
            pltpu.async_copy(r0, acc.at[pl.ds(sid * RPT + t * RB, RB)], zsem)
        for t in range(RPT // RB):
            pltpu.make_async_copy(
                r0, acc.at[pl.ds(sid * RPT + t * RB, RB)], zsem).wait()

        plsc.subcore_barrier()

        @pl.loop(0, NG)
        def _(gg):
            pltpu.sync_copy(src_hbm.at[wid, pl.ds(gg * GRP, GRP)], sidx)
            pltpu.sync_copy(dst_hbm.at[wid, pl.ds(gg * GRP, GRP)], didx)
            pltpu.sync_copy(ewp_hbm.at[wid, pl.ds(gg * GRP, GRP)], wbuf)

            gather(0, 0)
            gather(1, 1)

            @pl.loop(0, GRP // 4)
            def _(q):
                for j in range(4):
                    b = j
                    i = q * 4 + j
                    gather_wait(i, b)

                    @pl.loop(0, RB // 16)
                    def _(cv):
                        off = pl.multiple_of(cv * 16, 16)
                        w16 = wbuf[i, pl.ds(off, 16)]
                        for l in range(16):
                            wv = jnp.full((16,), w16[l], jnp.float32)
                            r = cv * 16 + l
                            for jj in range(D // 16):
                                rows[b][r, pl.ds(jj * 16, 16)] = (
                                    rows[b][r, pl.ds(jj * 16, 16)] * wv)

                    scatter(i, b)

                    b2 = (j + 2) % 4
                    if j < 2:
                        @pl.when(q > 0)
                        def _():
                            scatter_wait(q * 4 + j - 2, b2)
                        gather(i + 2, b2)
                    else:
                        scatter_wait(i - 2, b2)

                        @pl.when(q < GRP // 4 - 1)
                        def _():
                            gather(i + 2, b2)

            scatter_wait(GRP - 2, (GRP - 2) % 4)
            scatter_wait(GRP - 1, (GRP - 1) % 4)

        plsc.subcore_barrier()

        pltpu.sync_copy(acc.at[pl.ds(sid * RPT, RPT)],
                        out_hbm.at[cid, pl.ds(sid * RPT, RPT)])

    return k(ys, src3d, dst3d, ewp3d)


_ROWS_BLK = 1000
_GRID = N // _ROWS_BLK


def _rows_spec():
    return pl.BlockSpec((_ROWS_BLK, D), lambda i: (i, 0))


def _full_spec(shape):
    return pl.BlockSpec(shape, lambda i: tuple(0 for _ in shape))


def _tc_dis(degT):
    def body(dp_ref, o_ref):
        deg = dp_ref[...][:, 0:1] + dp_ref[...][:, 1:2] + 1.0
        o_ref[...] = jnp.broadcast_to(lax.rsqrt(deg), o_ref.shape)

    return pl.pallas_call(
        body,
        grid=(_GRID,),
        in_specs=[pl.BlockSpec((_ROWS_BLK, 2), lambda i: (i, 0))],
        out_specs=_rows_spec(),
        out_shape=jax.ShapeDtypeStruct((N, D), jnp.float32),
    )(degT)


def _tc_first(x, W, dis):
    def body(x_ref, w_ref, dis_ref, o_ref):
        o_ref[...] = dis_ref[...] * jnp.dot(
            x_ref[...], w_ref[...], preferred_element_type=jnp.float32)

    return pl.pallas_call(
        body,
        grid=(_GRID,),
        in_specs=[_rows_spec(), _full_spec((D, D)), _rows_spec()],
        out_specs=_rows_spec(),
        out_shape=jax.ShapeDtypeStruct((N, D), jnp.float32),
    )(x, W, dis)


def _tc_mid(p0, p1, ys, dis, b, W):
    def body(p0_ref, p1_ref, ys_ref, dis_ref, b_ref, w_ref, o_ref):
        t = dis_ref[...] * (p0_ref[...] + p1_ref[...] + ys_ref[...]) + b_ref[...]
        h = jnp.where(t > 0, t, jnp.exp(t) - 1.0)
        o_ref[...] = dis_ref[...] * jnp.dot(
            h, w_ref[...], preferred_element_type=jnp.float32)

    return pl.pallas_call(
        body,
        grid=(_GRID,),
        in_specs=[_rows_spec(), _rows_spec(), _rows_spec(), _rows_spec(),
                  _full_spec((1, D)), _full_spec((D, D))],
        out_specs=_rows_spec(),
        out_shape=jax.ShapeDtypeStruct((N, D), jnp.float32),
    )(p0, p1, ys, dis, b, W)


def _tc_last(p0, p1, ys, dis, b):
    def body(p0_ref, p1_ref, ys_ref, dis_ref, b_ref, o_ref):
        t = dis_ref[...] * (p0_ref[...] + p1_ref[...] + ys_ref[...]) + b_ref[...]
        nrm = jnp.sqrt(jnp.sum(t * t, axis=1, keepdims=True))
        o_ref[...] = t / jnp.maximum(nrm, 1e-12)

    return pl.pallas_call(
        body,
        grid=(_GRID,),
        in_specs=[_rows_spec(), _rows_spec(), _rows_spec(), _rows_spec(),
                  _full_spec((1, D))],
        out_specs=_rows_spec(),
        out_shape=jax.ShapeDtypeStruct((N, D), jnp.float32),
    )(p0, p1, ys, dis, b)


def kernel(x, edge_index, edge_weight, W1, b1, W2, b2, W3, b3):
    src3d = edge_index[0].reshape(NW, NCHT, C)
    dst3d = edge_index[1].reshape(NW, NCHT, C)
    ew3d = edge_weight.reshape(NW, NCHT, C)
    ewp3d = jnp.zeros((NW, NCHT, WPAD), jnp.float32).at[:, :, :C].set(ew3d)

    degp = _sc_degree(edge_index[1].reshape(NW, DNCH, DC),
                      edge_weight.reshape(NW, DNCH, DC)).reshape(NC, NP)
    dis = _tc_dis(degp.T)

    ys = _tc_first(x, W1, dis)
    for b, Wn in ((b1, W2), (b2, W3)):
        p = _sc_scatter(ys, src3d, dst3d, ewp3d)
        ys = _tc_mid(p[0], p[1], ys, dis, b.reshape(1, D), Wn)
    p = _sc_scatter(ys, src3d, dst3d, ewp3d)
    return _tc_last(p[0], p[1], ys, dis, b3.reshape(1, D))
